# Initial kernel scaffold; baseline (speedup 1.0000x reference)
import jax
import jax.numpy as jnp
from jax.experimental import pallas as pl


def kernel(x, edge_index, koff, W0, g0, b0, W1, g1, b1, W2, g2, b2):
    raise NotImplementedError("write your pallas kernel here")


# speedup vs baseline: 19.6849x; 19.6849x over previous
import functools

import jax
import jax.numpy as jnp
from jax import lax
from jax.experimental import pallas as pl
from jax.experimental.pallas import tpu as pltpu
import jax.experimental.pallas.tpu_sc as plsc

N = 50000
KV = 27
NC = 2
NS = 16
LANES = 16
CH = 128
NCHUNK = 391
EPT = NCHUNK * CH
EPAD = NS * EPT
ROWS_PT = 3128
ACC_ROWS = NS * ROWS_PT


def _matmul_tc(h, W, c_half, k_pad):
    C_in = h.shape[1]
    C_out = W.shape[2]
    Bn = 1000
    minor = k_pad * c_half

    def body(h_ref, w_ref, out_ref):
        hb = h_ref[...]
        for k in range(KV):
            r = jnp.dot(hb, w_ref[k], preferred_element_type=jnp.float32)
            out_ref[0, :, k * c_half:(k + 1) * c_half] = r[:, :c_half]
            out_ref[1, :, k * c_half:(k + 1) * c_half] = r[:, c_half:]
        pad = jnp.zeros((Bn, (k_pad - KV) * c_half), jnp.float32)
        out_ref[0, :, KV * c_half:] = pad
        out_ref[1, :, KV * c_half:] = pad

    return pl.pallas_call(
        body,
        grid=(N // Bn,),
        in_specs=[
            pl.BlockSpec((Bn, C_in), lambda i: (i, 0)),
            pl.BlockSpec((KV, C_in, C_out), lambda i: (0, 0, 0)),
        ],
        out_specs=pl.BlockSpec((NC, Bn, minor), lambda i: (0, i, 0)),
        out_shape=jax.ShapeDtypeStruct((NC, N, minor), jnp.float32),
    )(h, W)


def _edge_sc(table, src3, dst3, koff3, zeros, c_half, k_pad):
    mesh = plsc.VectorSubcoreMesh(core_axis_name="c", subcore_axis_name="s")

    @functools.partial(
        pl.kernel,
        out_type=jax.ShapeDtypeStruct((NC, ACC_ROWS, c_half), jnp.float32),
        mesh=mesh,
        compiler_params=pltpu.CompilerParams(use_tc_tiling_on_sc=False),
        scratch_types=[
            pltpu.VMEM((NCHUNK, CH), jnp.int32),
            pltpu.VMEM((NCHUNK, CH), jnp.int32),
            pltpu.VMEM((1, CH), jnp.int32),
            pltpu.VMEM((CH, c_half), jnp.float32),
            pltpu.VMEM_SHARED((ACC_ROWS, c_half), jnp.float32),
            pltpu.SemaphoreType.DMA,
        ],
    )
    def k(table_r, src_r, dst_r, koff_r, zero_r, out_r,
          idx_s, koff_s, dst_c, rows, acc, sem):
        cid = lax.axis_index("c")
        sid = lax.axis_index("s")
        rbase = sid * ROWS_PT

        pltpu.sync_copy(src_r.at[sid], idx_s)
        pltpu.sync_copy(koff_r.at[sid], koff_s)
        pltpu.sync_copy(zero_r.at[pl.ds(rbase, ROWS_PT)],
                        acc.at[pl.ds(rbase, ROWS_PT)])

        base = cid * N

        def cbody(i, carry):
            j = i // (CH // LANES)
            c = (i % (CH // LANES)) * LANES
            s = idx_s[j, pl.ds(c, LANES)]
            kf = koff_s[j, pl.ds(c, LANES)]
            idx_s[j, pl.ds(c, LANES)] = (s + base) * k_pad + kf
            return carry

        lax.fori_loop(0, NCHUNK * (CH // LANES), cbody, 0)
        plsc.subcore_barrier()

        def mbody(j, carry):
            pltpu.sync_copy(dst_r.at[sid, j], dst_c.at[0])
            pltpu.async_copy(table_r.at[idx_s.at[j]], rows, sem).wait()
            pltpu.sync_copy(rows, acc.at[dst_c.at[0]], add=True)
            return carry

        lax.fori_loop(0, NCHUNK, mbody, 0)
        plsc.subcore_barrier()
        pltpu.sync_copy(acc.at[pl.ds(rbase, ROWS_PT)],
                        out_r.at[cid, pl.ds(rbase, ROWS_PT)])

    return k(table, src3, dst3, koff3, zeros)


def _bn_relu_tc(s, g, b, c_half, eps=1e-3):
    G = 128 // c_half
    R = ACC_ROWS * c_half // 128
    Rn = N * c_half // 128
    s128 = s.reshape(NC, R, 128)
    g128 = jnp.tile(g.reshape(NC, c_half), (1, G)).reshape(NC, 1, 128)
    b128 = jnp.tile(b.reshape(NC, c_half), (1, G)).reshape(NC, 1, 128)

    def body(s_ref, g_ref, b_ref, out_ref):
        for cp in range(NC):
            hb = s_ref[cp, :Rn, :]
            m = jnp.mean(hb, axis=0, keepdims=True)
            mq = jnp.mean(hb * hb, axis=0, keepdims=True)
            mg = sum(m[:, i * c_half:(i + 1) * c_half] for i in range(G)) / G
            mqg = sum(mq[:, i * c_half:(i + 1) * c_half] for i in range(G)) / G
            inv = lax.rsqrt(mqg - mg * mg + eps)
            mt = jnp.concatenate([mg] * G, axis=1)
            invt = jnp.concatenate([inv] * G, axis=1)
            y = (s_ref[cp] - mt) * (invt * g_ref[cp]) + b_ref[cp]
            out_ref[cp] = jnp.maximum(y, 0.0)

    y128 = pl.pallas_call(
        body,
        in_specs=[
            pl.BlockSpec((NC, R, 128), lambda: (0, 0, 0)),
            pl.BlockSpec((NC, 1, 128), lambda: (0, 0, 0)),
            pl.BlockSpec((NC, 1, 128), lambda: (0, 0, 0)),
        ],
        out_specs=pl.BlockSpec((NC, R, 128), lambda: (0, 0, 0)),
        out_shape=jax.ShapeDtypeStruct((NC, R, 128), jnp.float32),
    )(s128, g128, b128)
    y = y128.reshape(NC, ACC_ROWS, c_half)
    return jnp.concatenate([y[0, :N], y[1, :N]], axis=1)


def kernel(x, edge_index, koff, W0, g0, b0, W1, g1, b1, W2, g2, b2):
    src = edge_index[0]
    dst = edge_index[1]
    pad = EPAD - src.shape[0]
    ar = jnp.arange(pad, dtype=jnp.int32)
    src3 = jnp.concatenate([src, ar % 1024]).reshape(NS, NCHUNK, CH)
    dst3 = jnp.concatenate([dst, N + (ar % (ACC_ROWS - N))]).reshape(NS, NCHUNK, CH)
    koff3 = jnp.concatenate([koff, ar * 0]).reshape(NS, NCHUNK, CH)

    h = x
    for (W, g, b) in ((W0, g0, b0), (W1, g1, b1), (W2, g2, b2)):
        C_out = W.shape[2]
        gw = min(C_out, 16)
        outs = []
        for c0 in range(0, C_out, gw):
            c_half = gw // 2
            k_pad = 128 // c_half * ((KV * c_half + 127) // 128)
            table = _matmul_tc(h, W[:, :, c0:c0 + gw], c_half, k_pad)
            table = table.reshape(NC * N * k_pad, c_half)
            zeros = jnp.zeros((ACC_ROWS, c_half), jnp.float32)
            s = _edge_sc(table, src3, dst3, koff3, zeros, c_half, k_pad)
            outs.append(_bn_relu_tc(s, g[c0:c0 + gw], b[c0:c0 + gw], c_half))
        h = outs[0] if len(outs) == 1 else jnp.concatenate(outs, axis=1)
    return h


# speedup vs baseline: 30.5969x; 1.5543x over previous
import functools

import jax
import jax.numpy as jnp
from jax import lax
from jax.experimental import pallas as pl
from jax.experimental.pallas import tpu as pltpu
import jax.experimental.pallas.tpu_sc as plsc

N = 50000
KV = 27
K_PAD = 32
GW = 16
GC_HALF = GW // 2
NC = 2
NS = 16
LANES = 16
CH = 128
NCHUNK = 392
EPT = NCHUNK * CH
EPAD = NS * EPT
ROWS_PT = 3128
ACC_ROWS = NS * ROWS_PT


def _matmul_tc(h, W):
    C_in = h.shape[1]
    Bn = 1000
    minor = K_PAD * GC_HALF

    def body(h_ref, w_ref, out_ref):
        hb = h_ref[...]
        for k in range(KV):
            r = jnp.dot(hb, w_ref[k], preferred_element_type=jnp.float32)
            out_ref[0, :, k * GC_HALF:(k + 1) * GC_HALF] = r[:, :GC_HALF]
            out_ref[1, :, k * GC_HALF:(k + 1) * GC_HALF] = r[:, GC_HALF:]
        pad = jnp.zeros((Bn, (K_PAD - KV) * GC_HALF), jnp.float32)
        out_ref[0, :, KV * GC_HALF:] = pad
        out_ref[1, :, KV * GC_HALF:] = pad

    return pl.pallas_call(
        body,
        grid=(N // Bn,),
        in_specs=[
            pl.BlockSpec((Bn, C_in), lambda i: (i, 0)),
            pl.BlockSpec((KV, C_in, GW), lambda i: (0, 0, 0)),
        ],
        out_specs=pl.BlockSpec((NC, Bn, minor), lambda i: (0, i, 0)),
        out_shape=jax.ShapeDtypeStruct((NC, N, minor), jnp.float32),
    )(h, W)


def _edge_sc(table, eidx3, dst3, zeros):
    mesh = plsc.VectorSubcoreMesh(core_axis_name="c", subcore_axis_name="s")

    @functools.partial(
        pl.kernel,
        out_type=jax.ShapeDtypeStruct((NC, ACC_ROWS, GC_HALF), jnp.float32),
        mesh=mesh,
        compiler_params=pltpu.CompilerParams(use_tc_tiling_on_sc=False),
        scratch_types=[
            pltpu.VMEM((NCHUNK, CH), jnp.int32),
            pltpu.VMEM((NCHUNK, CH), jnp.int32),
            pltpu.VMEM((2, CH, GC_HALF), jnp.float32),
            pltpu.VMEM_SHARED((ACC_ROWS, GC_HALF), jnp.float32),
            pltpu.SemaphoreType.DMA,
            pltpu.SemaphoreType.DMA,
            pltpu.SemaphoreType.DMA,
            pltpu.SemaphoreType.DMA,
        ],
    )
    def k(table_r, eidx_r, dst_r, zero_r, out_r,
          idx_s, dst_s, rows2, acc, gs0, gs1, ss0, ss1):
        cid = lax.axis_index("c")
        sid = lax.axis_index("s")
        rbase = sid * ROWS_PT
        tbl = table_r.at[cid]
        gsem = (gs0, gs1)
        ssem = (ss0, ss1)

        pltpu.sync_copy(eidx_r.at[sid], idx_s)
        pltpu.sync_copy(dst_r.at[sid], dst_s)
        pltpu.sync_copy(zero_r.at[pl.ds(rbase, ROWS_PT)],
                        acc.at[pl.ds(rbase, ROWS_PT)])
        plsc.subcore_barrier()

        def start_gather(j, p):
            pltpu.async_copy(tbl.at[idx_s.at[j]], rows2.at[p], gsem[p])

        def wait_gather(j, p):
            pltpu.make_async_copy(tbl.at[idx_s.at[j]], rows2.at[p],
                                  gsem[p]).wait()

        def start_scatter(j, p):
            pltpu.async_copy(rows2.at[p], acc.at[dst_s.at[j]], ssem[p],
                             add=True)

        def wait_scatter(j, p):
            pltpu.make_async_copy(rows2.at[p], acc.at[dst_s.at[j]],
                                  ssem[p]).wait()

        start_gather(0, 0)

        def step(j, p):
            q = 1 - p

            @pl.when(j + 1 < NCHUNK)
            def _():
                @pl.when(j >= 1)
                def _():
                    wait_scatter(j - 1, q)
                start_gather(j + 1, q)

            wait_gather(j, p)
            start_scatter(j, p)

        def mbody(j, carry):
            @pl.when(j % 2 == 0)
            def _():
                step(j, 0)

            @pl.when(j % 2 == 1)
            def _():
                step(j, 1)

            return carry

        lax.fori_loop(0, NCHUNK, mbody, 0)
        wait_scatter(NCHUNK - 2, 0)
        wait_scatter(NCHUNK - 1, 1)
        plsc.subcore_barrier()
        pltpu.sync_copy(acc.at[pl.ds(rbase, ROWS_PT)],
                        out_r.at[cid, pl.ds(rbase, ROWS_PT)])

    return k(table, eidx3, dst3, zeros)


def _bn_relu_tc(s, g, b, eps=1e-3):
    c_half = GC_HALF
    G = 128 // c_half
    R = ACC_ROWS * c_half // 128
    Rn = N * c_half // 128
    s128 = s.reshape(NC, R, 128)
    g128 = jnp.tile(g.reshape(NC, c_half), (1, G)).reshape(NC, 1, 128)
    b128 = jnp.tile(b.reshape(NC, c_half), (1, G)).reshape(NC, 1, 128)

    def body(s_ref, g_ref, b_ref, out_ref):
        for cp in range(NC):
            hb = s_ref[cp, :Rn, :]
            m = jnp.mean(hb, axis=0, keepdims=True)
            mq = jnp.mean(hb * hb, axis=0, keepdims=True)
            mg = sum(m[:, i * c_half:(i + 1) * c_half] for i in range(G)) / G
            mqg = sum(mq[:, i * c_half:(i + 1) * c_half] for i in range(G)) / G
            inv = lax.rsqrt(mqg - mg * mg + eps)
            mt = jnp.concatenate([mg] * G, axis=1)
            invt = jnp.concatenate([inv] * G, axis=1)
            y = (s_ref[cp] - mt) * (invt * g_ref[cp]) + b_ref[cp]
            out_ref[cp] = jnp.maximum(y, 0.0)

    y128 = pl.pallas_call(
        body,
        in_specs=[
            pl.BlockSpec((NC, R, 128), lambda: (0, 0, 0)),
            pl.BlockSpec((NC, 1, 128), lambda: (0, 0, 0)),
            pl.BlockSpec((NC, 1, 128), lambda: (0, 0, 0)),
        ],
        out_specs=pl.BlockSpec((NC, R, 128), lambda: (0, 0, 0)),
        out_shape=jax.ShapeDtypeStruct((NC, R, 128), jnp.float32),
    )(s128, g128, b128)
    y = y128.reshape(NC, ACC_ROWS, c_half)
    return jnp.concatenate([y[0, :N], y[1, :N]], axis=1)


def kernel(x, edge_index, koff, W0, g0, b0, W1, g1, b1, W2, g2, b2):
    src = edge_index[0]
    dst = edge_index[1]
    pad = EPAD - src.shape[0]
    ar = jnp.arange(pad, dtype=jnp.int32)
    src_p = jnp.concatenate([src, ar % 1024])
    koff_p = jnp.concatenate([koff, ar * 0])
    eidx3 = (src_p * K_PAD + koff_p).reshape(NS, NCHUNK, CH)
    dst3 = jnp.concatenate([dst, N + (ar % (ACC_ROWS - N))]).reshape(NS, NCHUNK, CH)
    zeros = jnp.zeros((ACC_ROWS, GC_HALF), jnp.float32)

    h = x
    for (W, g, b) in ((W0, g0, b0), (W1, g1, b1), (W2, g2, b2)):
        C_out = W.shape[2]
        outs = []
        for c0 in range(0, C_out, GW):
            table = _matmul_tc(h, W[:, :, c0:c0 + GW])
            s = _edge_sc(table.reshape(NC, N * K_PAD, GC_HALF), eidx3, dst3,
                         zeros)
            outs.append(_bn_relu_tc(s, g[c0:c0 + GW], b[c0:c0 + GW]))
        h = outs[0] if len(outs) == 1 else jnp.concatenate(outs, axis=1)
    return h


# speedup vs baseline: 30.6219x; 1.0008x over previous
import functools

import jax
import jax.numpy as jnp
from jax import lax
from jax.experimental import pallas as pl
from jax.experimental.pallas import tpu as pltpu
import jax.experimental.pallas.tpu_sc as plsc

N = 50000
KV = 27
K_PAD = 32
GW = 16
GC_HALF = GW // 2
NC = 2
NS = 16
LANES = 16
CH = 128
NCHUNK = 392
EPT = NCHUNK * CH
EPAD = NS * EPT
ROWS_PT = 3128
ACC_ROWS = NS * ROWS_PT


def _matmul_tc(h, W, c_half):
    C_in = h.shape[1]
    gw = 2 * c_half
    Bn = 1000
    minor = K_PAD * c_half

    def body(h_ref, w_ref, out_ref):
        hb = h_ref[...]
        for k in range(KV):
            r = jnp.dot(hb, w_ref[k], preferred_element_type=jnp.float32)
            out_ref[0, :, k * c_half:(k + 1) * c_half] = r[:, :c_half]
            out_ref[1, :, k * c_half:(k + 1) * c_half] = r[:, c_half:]
        pad = jnp.zeros((Bn, (K_PAD - KV) * c_half), jnp.float32)
        out_ref[0, :, KV * c_half:] = pad
        out_ref[1, :, KV * c_half:] = pad

    return pl.pallas_call(
        body,
        grid=(N // Bn,),
        in_specs=[
            pl.BlockSpec((Bn, C_in), lambda i: (i, 0)),
            pl.BlockSpec((KV, C_in, gw), lambda i: (0, 0, 0)),
        ],
        out_specs=pl.BlockSpec((NC, Bn, minor), lambda i: (0, i, 0)),
        out_shape=jax.ShapeDtypeStruct((NC, N, minor), jnp.float32),
    )(h, W)


def _edge_sc(table, eidx3, dst3, zeros, c_half):
    mesh = plsc.VectorSubcoreMesh(core_axis_name="c", subcore_axis_name="s")

    @functools.partial(
        pl.kernel,
        out_type=jax.ShapeDtypeStruct((NC, ACC_ROWS, c_half), jnp.float32),
        mesh=mesh,
        compiler_params=pltpu.CompilerParams(use_tc_tiling_on_sc=False),
        scratch_types=[
            pltpu.VMEM((NCHUNK, CH), jnp.int32),
            pltpu.VMEM((NCHUNK, CH), jnp.int32),
            pltpu.VMEM((2, CH, c_half), jnp.float32),
            pltpu.VMEM_SHARED((ACC_ROWS, c_half), jnp.float32),
            pltpu.SemaphoreType.DMA,
            pltpu.SemaphoreType.DMA,
            pltpu.SemaphoreType.DMA,
            pltpu.SemaphoreType.DMA,
        ],
    )
    def k(table_r, eidx_r, dst_r, zero_r, out_r,
          idx_s, dst_s, rows2, acc, gs0, gs1, ss0, ss1):
        cid = lax.axis_index("c")
        sid = lax.axis_index("s")
        rbase = sid * ROWS_PT
        tbl = table_r.at[cid]
        gsem = (gs0, gs1)
        ssem = (ss0, ss1)

        pltpu.sync_copy(eidx_r.at[sid], idx_s)
        pltpu.sync_copy(dst_r.at[sid], dst_s)
        pltpu.sync_copy(zero_r.at[pl.ds(rbase, ROWS_PT)],
                        acc.at[pl.ds(rbase, ROWS_PT)])
        plsc.subcore_barrier()

        def start_gather(j, p):
            pltpu.async_copy(tbl.at[idx_s.at[j]], rows2.at[p], gsem[p])

        def wait_gather(j, p):
            pltpu.make_async_copy(tbl.at[idx_s.at[j]], rows2.at[p],
                                  gsem[p]).wait()

        def start_scatter(j, p):
            pltpu.async_copy(rows2.at[p], acc.at[dst_s.at[j]], ssem[p],
                             add=True)

        def wait_scatter(j, p):
            pltpu.make_async_copy(rows2.at[p], acc.at[dst_s.at[j]],
                                  ssem[p]).wait()

        start_gather(0, 0)

        def step(j, p):
            q = 1 - p

            @pl.when(j + 1 < NCHUNK)
            def _():
                @pl.when(j >= 1)
                def _():
                    wait_scatter(j - 1, q)
                start_gather(j + 1, q)

            wait_gather(j, p)
            start_scatter(j, p)

        def mbody(j, carry):
            @pl.when(j % 2 == 0)
            def _():
                step(j, 0)

            @pl.when(j % 2 == 1)
            def _():
                step(j, 1)

            return carry

        lax.fori_loop(0, NCHUNK, mbody, 0)
        wait_scatter(NCHUNK - 2, 0)
        wait_scatter(NCHUNK - 1, 1)
        plsc.subcore_barrier()
        pltpu.sync_copy(acc.at[pl.ds(rbase, ROWS_PT)],
                        out_r.at[cid, pl.ds(rbase, ROWS_PT)])

    return k(table, eidx3, dst3, zeros)


def _bn_relu_tc(s, g, b, c_half, eps=1e-3):
    G = 128 // c_half
    R = ACC_ROWS * c_half // 128
    Rn = N * c_half // 128
    s128 = s.reshape(NC, R, 128)
    g128 = jnp.tile(g.reshape(NC, c_half), (1, G)).reshape(NC, 1, 128)
    b128 = jnp.tile(b.reshape(NC, c_half), (1, G)).reshape(NC, 1, 128)

    def body(s_ref, g_ref, b_ref, out_ref):
        for cp in range(NC):
            hb = s_ref[cp, :Rn, :]
            m = jnp.mean(hb, axis=0, keepdims=True)
            mq = jnp.mean(hb * hb, axis=0, keepdims=True)
            mg = sum(m[:, i * c_half:(i + 1) * c_half] for i in range(G)) / G
            mqg = sum(mq[:, i * c_half:(i + 1) * c_half] for i in range(G)) / G
            inv = lax.rsqrt(mqg - mg * mg + eps)
            mt = jnp.concatenate([mg] * G, axis=1)
            invt = jnp.concatenate([inv] * G, axis=1)
            y = (s_ref[cp] - mt) * (invt * g_ref[cp]) + b_ref[cp]
            out_ref[cp] = jnp.maximum(y, 0.0)

    y128 = pl.pallas_call(
        body,
        in_specs=[
            pl.BlockSpec((NC, R, 128), lambda: (0, 0, 0)),
            pl.BlockSpec((NC, 1, 128), lambda: (0, 0, 0)),
            pl.BlockSpec((NC, 1, 128), lambda: (0, 0, 0)),
        ],
        out_specs=pl.BlockSpec((NC, R, 128), lambda: (0, 0, 0)),
        out_shape=jax.ShapeDtypeStruct((NC, R, 128), jnp.float32),
    )(s128, g128, b128)
    y = y128.reshape(NC, ACC_ROWS, c_half)
    return jnp.concatenate([y[0, :N], y[1, :N]], axis=1)


def kernel(x, edge_index, koff, W0, g0, b0, W1, g1, b1, W2, g2, b2):
    src = edge_index[0]
    dst = edge_index[1]
    pad = EPAD - src.shape[0]
    ar = jnp.arange(pad, dtype=jnp.int32)
    src_p = jnp.concatenate([src, ar % 1024])
    koff_p = jnp.concatenate([koff, ar * 0])
    eidx3 = (src_p * K_PAD + koff_p).reshape(NS, NCHUNK, CH)
    dst3 = jnp.concatenate([dst, N + (ar % (ACC_ROWS - N))]).reshape(NS, NCHUNK, CH)
    h = x
    for (W, g, b) in ((W0, g0, b0), (W1, g1, b1), (W2, g2, b2)):
        C_out = W.shape[2]
        gw = min(GW, C_out)
        c_half = gw // 2
        outs = []
        for c0 in range(0, C_out, gw):
            table = _matmul_tc(h, W[:, :, c0:c0 + gw], c_half)
            s = _edge_sc(table.reshape(NC, N * K_PAD, c_half), eidx3, dst3,
                         jnp.zeros((ACC_ROWS, c_half), jnp.float32), c_half)
            outs.append(_bn_relu_tc(s, g[c0:c0 + gw], b[c0:c0 + gw], c_half))
        h = outs[0] if len(outs) == 1 else jnp.concatenate(outs, axis=1)
    return h


# speedup vs baseline: 31.3320x; 1.0232x over previous
import functools

import jax
import jax.numpy as jnp
from jax import lax
from jax.experimental import pallas as pl
from jax.experimental.pallas import tpu as pltpu
import jax.experimental.pallas.tpu_sc as plsc

N = 50000
KV = 27
K_PAD = 32
GW = 16
GC_HALF = GW // 2
NC = 2
NS = 16
LANES = 16
CH = 128
NCHUNK = 392
EPT = NCHUNK * CH
EPAD = NS * EPT
ROWS_PT = 3128
ACC_ROWS = NS * ROWS_PT


def _matmul_tc(h, W, c_half):
    C_in = h.shape[1]
    gw = 2 * c_half
    Bn = 1000
    minor = K_PAD * c_half

    def body(h_ref, w_ref, out_ref):
        hb = h_ref[...]
        for k in range(KV):
            r = jnp.dot(hb, w_ref[k], preferred_element_type=jnp.float32)
            out_ref[0, :, k * c_half:(k + 1) * c_half] = r[:, :c_half]
            out_ref[1, :, k * c_half:(k + 1) * c_half] = r[:, c_half:]
        pad = jnp.zeros((Bn, (K_PAD - KV) * c_half), jnp.float32)
        out_ref[0, :, KV * c_half:] = pad
        out_ref[1, :, KV * c_half:] = pad

    return pl.pallas_call(
        body,
        grid=(N // Bn,),
        in_specs=[
            pl.BlockSpec((Bn, C_in), lambda i: (i, 0)),
            pl.BlockSpec((KV, C_in, gw), lambda i: (0, 0, 0)),
        ],
        out_specs=pl.BlockSpec((NC, Bn, minor), lambda i: (0, i, 0)),
        out_shape=jax.ShapeDtypeStruct((NC, N, minor), jnp.float32),
    )(h, W)


def _edge_sc(table, eidx3, dst3, zeros, c_half):
    mesh = plsc.VectorSubcoreMesh(core_axis_name="c", subcore_axis_name="s")

    @functools.partial(
        pl.kernel,
        out_type=jax.ShapeDtypeStruct((NC, ACC_ROWS, c_half), jnp.float32),
        mesh=mesh,
        compiler_params=pltpu.CompilerParams(use_tc_tiling_on_sc=False),
        scratch_types=[
            pltpu.VMEM((NCHUNK, CH), jnp.int32),
            pltpu.VMEM((NCHUNK, CH), jnp.int32),
            pltpu.VMEM((4, CH, c_half), jnp.float32),
            pltpu.VMEM_SHARED((ACC_ROWS, c_half), jnp.float32),
            pltpu.SemaphoreType.DMA,
            pltpu.SemaphoreType.DMA,
            pltpu.SemaphoreType.DMA,
            pltpu.SemaphoreType.DMA,
            pltpu.SemaphoreType.DMA,
            pltpu.SemaphoreType.DMA,
            pltpu.SemaphoreType.DMA,
            pltpu.SemaphoreType.DMA,
        ],
    )
    def k(table_r, eidx_r, dst_r, zero_r, out_r,
          idx_s, dst_s, rows2, acc, gs0, gs1, gs2, gs3, ss0, ss1, ss2, ss3):
        cid = lax.axis_index("c")
        sid = lax.axis_index("s")
        rbase = sid * ROWS_PT
        tbl = table_r.at[cid]
        gsem = (gs0, gs1, gs2, gs3)
        ssem = (ss0, ss1, ss2, ss3)

        pltpu.sync_copy(eidx_r.at[sid], idx_s)
        pltpu.sync_copy(dst_r.at[sid], dst_s)
        pltpu.sync_copy(zero_r.at[pl.ds(rbase, ROWS_PT)],
                        acc.at[pl.ds(rbase, ROWS_PT)])
        plsc.subcore_barrier()

        def start_gather(j, p):
            pltpu.async_copy(tbl.at[idx_s.at[j]], rows2.at[p], gsem[p])

        def wait_gather(j, p):
            pltpu.make_async_copy(tbl.at[idx_s.at[j]], rows2.at[p],
                                  gsem[p]).wait()

        def start_scatter(j, p):
            pltpu.async_copy(rows2.at[p], acc.at[dst_s.at[j]], ssem[p],
                             add=True)

        def wait_scatter(j, p):
            pltpu.make_async_copy(rows2.at[p], acc.at[dst_s.at[j]],
                                  ssem[p]).wait()

        start_gather(0, 0)

        def step(j, p):
            q = (p + 1) % 4

            @pl.when(j + 1 < NCHUNK)
            def _():
                @pl.when(j >= 3)
                def _():
                    wait_scatter(j - 3, q)
                start_gather(j + 1, q)

            wait_gather(j, p)
            start_scatter(j, p)

        def mbody(j, carry):
            for ph in range(4):
                @pl.when(j % 4 == ph)
                def _(ph=ph):
                    step(j, ph)

            return carry

        lax.fori_loop(0, NCHUNK, mbody, 0)
        for tail in range(4):
            j = NCHUNK - 4 + tail
            wait_scatter(j, j % 4)
        plsc.subcore_barrier()
        pltpu.sync_copy(acc.at[pl.ds(rbase, ROWS_PT)],
                        out_r.at[cid, pl.ds(rbase, ROWS_PT)])

    return k(table, eidx3, dst3, zeros)


def _bn_relu_tc(s, g, b, c_half, eps=1e-3):
    G = 128 // c_half
    R = ACC_ROWS * c_half // 128
    Rn = N * c_half // 128
    s128 = s.reshape(NC, R, 128)
    g128 = jnp.tile(g.reshape(NC, c_half), (1, G)).reshape(NC, 1, 128)
    b128 = jnp.tile(b.reshape(NC, c_half), (1, G)).reshape(NC, 1, 128)

    def body(s_ref, g_ref, b_ref, out_ref):
        for cp in range(NC):
            hb = s_ref[cp, :Rn, :]
            m = jnp.mean(hb, axis=0, keepdims=True)
            mq = jnp.mean(hb * hb, axis=0, keepdims=True)
            mg = sum(m[:, i * c_half:(i + 1) * c_half] for i in range(G)) / G
            mqg = sum(mq[:, i * c_half:(i + 1) * c_half] for i in range(G)) / G
            inv = lax.rsqrt(mqg - mg * mg + eps)
            mt = jnp.concatenate([mg] * G, axis=1)
            invt = jnp.concatenate([inv] * G, axis=1)
            y = (s_ref[cp] - mt) * (invt * g_ref[cp]) + b_ref[cp]
            out_ref[cp] = jnp.maximum(y, 0.0)

    y128 = pl.pallas_call(
        body,
        in_specs=[
            pl.BlockSpec((NC, R, 128), lambda: (0, 0, 0)),
            pl.BlockSpec((NC, 1, 128), lambda: (0, 0, 0)),
            pl.BlockSpec((NC, 1, 128), lambda: (0, 0, 0)),
        ],
        out_specs=pl.BlockSpec((NC, R, 128), lambda: (0, 0, 0)),
        out_shape=jax.ShapeDtypeStruct((NC, R, 128), jnp.float32),
    )(s128, g128, b128)
    y = y128.reshape(NC, ACC_ROWS, c_half)
    return jnp.concatenate([y[0, :N], y[1, :N]], axis=1)


def kernel(x, edge_index, koff, W0, g0, b0, W1, g1, b1, W2, g2, b2):
    src = edge_index[0]
    dst = edge_index[1]
    pad = EPAD - src.shape[0]
    ar = jnp.arange(pad, dtype=jnp.int32)
    src_p = jnp.concatenate([src, ar % 1024])
    koff_p = jnp.concatenate([koff, ar * 0])
    eidx3 = (src_p * K_PAD + koff_p).reshape(NS, NCHUNK, CH)
    dst3 = jnp.concatenate([dst, N + (ar % (ACC_ROWS - N))]).reshape(NS, NCHUNK, CH)
    h = x
    for (W, g, b) in ((W0, g0, b0), (W1, g1, b1), (W2, g2, b2)):
        C_out = W.shape[2]
        gw = min(GW, C_out)
        c_half = gw // 2
        outs = []
        for c0 in range(0, C_out, gw):
            table = _matmul_tc(h, W[:, :, c0:c0 + gw], c_half)
            s = _edge_sc(table.reshape(NC, N * K_PAD, c_half), eidx3, dst3,
                         jnp.zeros((ACC_ROWS, c_half), jnp.float32), c_half)
            outs.append(_bn_relu_tc(s, g[c0:c0 + gw], b[c0:c0 + gw], c_half))
        h = outs[0] if len(outs) == 1 else jnp.concatenate(outs, axis=1)
    return h


# speedup vs baseline: 33.5989x; 1.0724x over previous
import functools

import jax
import jax.numpy as jnp
from jax import lax
from jax.experimental import pallas as pl
from jax.experimental.pallas import tpu as pltpu
import jax.experimental.pallas.tpu_sc as plsc

N = 50000
KV = 27
K_PAD = 32
GW = 16
GC_HALF = GW // 2
NC = 2
NS = 16
LANES = 16
CH = 128
NCHUNK = 392
EPT = NCHUNK * CH
EPAD = NS * EPT
ROWS_PT = 3128
ACC_ROWS = NS * ROWS_PT


def _matmul_tc(h, W, c_half):
    C_in = h.shape[1]
    gw = 2 * c_half
    Bn = 1000
    minor = K_PAD * c_half

    def body(h_ref, w_ref, out_ref):
        hb = h_ref[...]
        for k in range(KV):
            r = jnp.dot(hb, w_ref[k], preferred_element_type=jnp.float32)
            out_ref[0, :, k * c_half:(k + 1) * c_half] = r[:, :c_half]
            out_ref[1, :, k * c_half:(k + 1) * c_half] = r[:, c_half:]
        pad = jnp.zeros((Bn, (K_PAD - KV) * c_half), jnp.float32)
        out_ref[0, :, KV * c_half:] = pad
        out_ref[1, :, KV * c_half:] = pad

    return pl.pallas_call(
        body,
        grid=(N // Bn,),
        in_specs=[
            pl.BlockSpec((Bn, C_in), lambda i: (i, 0)),
            pl.BlockSpec((KV, C_in, gw), lambda i: (0, 0, 0)),
        ],
        out_specs=pl.BlockSpec((NC, Bn, minor), lambda i: (0, i, 0)),
        out_shape=jax.ShapeDtypeStruct((NC, N, minor), jnp.float32),
    )(h, W)


def _edge_sc(table, eidx3, dst3, zeros, c_half):
    mesh = plsc.VectorSubcoreMesh(core_axis_name="c", subcore_axis_name="s")

    @functools.partial(
        pl.kernel,
        out_type=jax.ShapeDtypeStruct((NC, ACC_ROWS, c_half), jnp.float32),
        mesh=mesh,
        compiler_params=pltpu.CompilerParams(use_tc_tiling_on_sc=False),
        scratch_types=[
            pltpu.VMEM((NCHUNK, CH), jnp.int32),
            pltpu.VMEM((NCHUNK, CH), jnp.int32),
            pltpu.VMEM((4, CH, c_half), jnp.float32),
            pltpu.VMEM_SHARED((ACC_ROWS, c_half), jnp.float32),
            pltpu.SemaphoreType.DMA,
            pltpu.SemaphoreType.DMA,
            pltpu.SemaphoreType.DMA,
            pltpu.SemaphoreType.DMA,
            pltpu.SemaphoreType.DMA,
            pltpu.SemaphoreType.DMA,
            pltpu.SemaphoreType.DMA,
            pltpu.SemaphoreType.DMA,
        ],
    )
    def k(table_r, eidx_r, dst_r, zero_r, out_r,
          idx_s, dst_s, rows2, acc, gs0, gs1, gs2, gs3, ss0, ss1, ss2, ss3):
        cid = lax.axis_index("c")
        sid = lax.axis_index("s")
        rbase = sid * ROWS_PT
        tbl = table_r.at[cid]
        gsem = (gs0, gs1, gs2, gs3)
        ssem = (ss0, ss1, ss2, ss3)

        pltpu.sync_copy(eidx_r.at[sid], idx_s)
        pltpu.sync_copy(dst_r.at[sid], dst_s)
        pltpu.sync_copy(zero_r.at[pl.ds(rbase, ROWS_PT)],
                        acc.at[pl.ds(rbase, ROWS_PT)])
        plsc.subcore_barrier()

        def start_gather(j, p):
            pltpu.async_copy(tbl.at[idx_s.at[j]], rows2.at[p], gsem[p])

        def wait_gather(j, p):
            pltpu.make_async_copy(tbl.at[idx_s.at[j]], rows2.at[p],
                                  gsem[p]).wait()

        def start_scatter(j, p):
            pltpu.async_copy(rows2.at[p], acc.at[dst_s.at[j]], ssem[p],
                             add=True)

        def wait_scatter(j, p):
            pltpu.make_async_copy(rows2.at[p], acc.at[dst_s.at[j]],
                                  ssem[p]).wait()

        for j0 in range(2):
            start_gather(j0, j0)

        def step(j, p):
            q = (p + 2) % 4

            @pl.when(j + 2 < NCHUNK)
            def _():
                @pl.when(j >= 2)
                def _():
                    wait_scatter(j - 2, q)
                start_gather(j + 2, q)

            wait_gather(j, p)
            start_scatter(j, p)

        def mbody(j, carry):
            for ph in range(4):
                @pl.when(j % 4 == ph)
                def _(ph=ph):
                    step(j, ph)

            return carry

        lax.fori_loop(0, NCHUNK, mbody, 0)
        for tail in range(2):
            j = NCHUNK - 2 + tail
            wait_scatter(j, j % 4)
        plsc.subcore_barrier()
        pltpu.sync_copy(acc.at[pl.ds(rbase, ROWS_PT)],
                        out_r.at[cid, pl.ds(rbase, ROWS_PT)])

    return k(table, eidx3, dst3, zeros)


def _bn_relu_tc(s, g, b, c_half, eps=1e-3):
    G = 128 // c_half
    R = ACC_ROWS * c_half // 128
    Rn = N * c_half // 128
    s128 = s.reshape(NC, R, 128)
    g128 = jnp.tile(g.reshape(NC, c_half), (1, G)).reshape(NC, 1, 128)
    b128 = jnp.tile(b.reshape(NC, c_half), (1, G)).reshape(NC, 1, 128)

    def body(s_ref, g_ref, b_ref, out_ref):
        for cp in range(NC):
            hb = s_ref[cp, :Rn, :]
            m = jnp.mean(hb, axis=0, keepdims=True)
            mq = jnp.mean(hb * hb, axis=0, keepdims=True)
            mg = sum(m[:, i * c_half:(i + 1) * c_half] for i in range(G)) / G
            mqg = sum(mq[:, i * c_half:(i + 1) * c_half] for i in range(G)) / G
            inv = lax.rsqrt(mqg - mg * mg + eps)
            mt = jnp.concatenate([mg] * G, axis=1)
            invt = jnp.concatenate([inv] * G, axis=1)
            y = (s_ref[cp] - mt) * (invt * g_ref[cp]) + b_ref[cp]
            out_ref[cp] = jnp.maximum(y, 0.0)

    y128 = pl.pallas_call(
        body,
        in_specs=[
            pl.BlockSpec((NC, R, 128), lambda: (0, 0, 0)),
            pl.BlockSpec((NC, 1, 128), lambda: (0, 0, 0)),
            pl.BlockSpec((NC, 1, 128), lambda: (0, 0, 0)),
        ],
        out_specs=pl.BlockSpec((NC, R, 128), lambda: (0, 0, 0)),
        out_shape=jax.ShapeDtypeStruct((NC, R, 128), jnp.float32),
    )(s128, g128, b128)
    y = y128.reshape(NC, ACC_ROWS, c_half)
    return jnp.concatenate([y[0, :N], y[1, :N]], axis=1)


def kernel(x, edge_index, koff, W0, g0, b0, W1, g1, b1, W2, g2, b2):
    src = edge_index[0]
    dst = edge_index[1]
    pad = EPAD - src.shape[0]
    ar = jnp.arange(pad, dtype=jnp.int32)
    src_p = jnp.concatenate([src, ar % 1024])
    koff_p = jnp.concatenate([koff, ar * 0])
    eidx3 = (src_p * K_PAD + koff_p).reshape(NS, NCHUNK, CH)
    dst3 = jnp.concatenate([dst, N + (ar % (ACC_ROWS - N))]).reshape(NS, NCHUNK, CH)
    h = x
    for (W, g, b) in ((W0, g0, b0), (W1, g1, b1), (W2, g2, b2)):
        C_out = W.shape[2]
        gw = min(GW, C_out)
        c_half = gw // 2
        outs = []
        for c0 in range(0, C_out, gw):
            table = _matmul_tc(h, W[:, :, c0:c0 + gw], c_half)
            s = _edge_sc(table.reshape(NC, N * K_PAD, c_half), eidx3, dst3,
                         jnp.zeros((ACC_ROWS, c_half), jnp.float32), c_half)
            outs.append(_bn_relu_tc(s, g[c0:c0 + gw], b[c0:c0 + gw], c_half))
        h = outs[0] if len(outs) == 1 else jnp.concatenate(outs, axis=1)
    return h
